# baseline (device time: 19127 ns/iter reference)
import jax
import jax.numpy as jnp
from jax import lax
from jax.experimental import pallas as pl
from jax.experimental.pallas import tpu as pltpu

NB = 6


def kernel(A, B):
    m, k = A.shape
    _, n = B.shape
    half = m // 2
    qm = m // 4
    nc = n // NB

    def body(a_ref, b_ref, out_ref, ab_ref, bb_ref, ar_ref, br_ref,
             a_send, a_recv, af_send, af_recv, b_send, b_recv):
        my_x = lax.axis_index("x")
        my_y = lax.axis_index("y")
        p = (1 - my_x, my_y)
        q = (my_x, 1 - my_y)

        ab_ref[...] = a_ref[...].astype(jnp.bfloat16)
        bb_ref[...] = b_ref[...].astype(jnp.bfloat16)

        barrier_sem = pltpu.get_barrier_semaphore()
        for nbr in (p, q):
            pl.semaphore_signal(barrier_sem, inc=1, device_id=nbr,
                                device_id_type=pl.DeviceIdType.MESH)
        pl.semaphore_wait(barrier_sem, 2)

        def a_direct_rdma(row0, c):
            sl = slice(row0 + c * qm, row0 + (c + 1) * qm)
            return pltpu.make_async_remote_copy(
                src_ref=ab_ref.at[sl, :], dst_ref=ar_ref.at[sl, :],
                send_sem=a_send.at[c], recv_sem=a_recv.at[c],
                device_id=p, device_id_type=pl.DeviceIdType.MESH)

        def a_fwd_rdma(row0, c):
            sl = slice(row0 + c * qm, row0 + (c + 1) * qm)
            return pltpu.make_async_remote_copy(
                src_ref=ar_ref.at[sl, :], dst_ref=ar_ref.at[sl, :],
                send_sem=af_send.at[c], recv_sem=af_recv.at[c],
                device_id=q, device_id_type=pl.DeviceIdType.MESH)

        @pl.when(my_y == 0)
        def _():
            for c in range(2):
                a_direct_rdma(0, c).start()

        @pl.when(my_y == 1)
        def _():
            for c in range(2):
                a_direct_rdma(half, c).start()

        b_rdmas = []
        for j in range(NB):
            slc = slice(j * nc, (j + 1) * nc)
            r = pltpu.make_async_remote_copy(
                src_ref=bb_ref.at[:, slc], dst_ref=br_ref.at[:, slc],
                send_sem=b_send.at[j], recv_sem=b_recv.at[j],
                device_id=p, device_id_type=pl.DeviceIdType.MESH)
            r.start()
            b_rdmas.append(r)

        for c in range(2):
            a_direct_rdma(0, c).wait_recv()

            @pl.when(my_y == 0)
            def _(c=c):
                a_fwd_rdma(0, c).start()

            @pl.when(my_y == 1)
            def _(c=c):
                a_fwd_rdma(half, c).start()

        out_ref[...] = jnp.dot(ab_ref[...], bb_ref[...],
                               preferred_element_type=jnp.float32)

        for c in range(2):
            a_fwd_rdma(0, c).wait_recv()

        for j in range(NB):
            slc = slice(j * nc, (j + 1) * nc)
            b_rdmas[j].wait_recv()
            out_ref[:, slc] = out_ref[:, slc] + jnp.dot(
                ar_ref[...], br_ref[:, slc],
                preferred_element_type=jnp.float32)

        for c in range(2):
            a_direct_rdma(0, c).wait_send()
            a_fwd_rdma(0, c).wait_send()
        for r in b_rdmas:
            r.wait_send()

    return pl.pallas_call(
        body,
        out_shape=jax.ShapeDtypeStruct((m, n), jnp.float32),
        in_specs=[
            pl.BlockSpec(memory_space=pltpu.VMEM),
            pl.BlockSpec(memory_space=pltpu.VMEM),
        ],
        out_specs=pl.BlockSpec(memory_space=pltpu.VMEM),
        scratch_shapes=[
            pltpu.VMEM((m, k), jnp.bfloat16),
            pltpu.VMEM((k, n), jnp.bfloat16),
            pltpu.VMEM((m, k), jnp.bfloat16),
            pltpu.VMEM((k, n), jnp.bfloat16),
            pltpu.SemaphoreType.DMA((2,)),
            pltpu.SemaphoreType.DMA((2,)),
            pltpu.SemaphoreType.DMA((2,)),
            pltpu.SemaphoreType.DMA((2,)),
            pltpu.SemaphoreType.DMA((NB,)),
            pltpu.SemaphoreType.DMA((NB,)),
        ],
        compiler_params=pltpu.CompilerParams(collective_id=0),
    )(A, B)
